# Initial kernel scaffold; baseline (speedup 1.0000x reference)
import jax
import jax.numpy as jnp
from jax.experimental import pallas as pl


def kernel(seq, seq_length, emb, w_ih_f, w_hh_f, b_ih_f, b_hh_f, w_ih_b, w_hh_b, b_ih_b, b_hh_b, dec_W, dec_b):
    raise NotImplementedError("write your pallas kernel here")


# speedup vs baseline: 2.8921x; 2.8921x over previous
import functools

import jax
import jax.numpy as jnp
from jax.experimental import pallas as pl
from jax.experimental.pallas import tpu as pltpu

L = 50
B = 16
V = 100000
E = 32
H = 64
NROWS = (L - 2) * B
VB = 4096
NVB = (V + VB - 1) // VB

NEG_INF = float("-inf")


def _gru_kernel(x_ref, wihT_f_ref, whhT_f_ref, b_ih_f_ref, b_hh_f_ref,
                wihT_b_ref, whhT_b_ref, b_ih_b_ref, b_hh_b_ref, out_ref):
    wihT_f = wihT_f_ref[...]
    whhT_f = whhT_f_ref[...]
    b_f = b_ih_f_ref[...] + 0.0
    bh_f = b_hh_f_ref[...]
    wihT_b = wihT_b_ref[...]
    whhT_b = whhT_b_ref[...]
    b_b = b_ih_b_ref[...]
    bh_b = b_hh_b_ref[...]

    def cell(xt, h, wihT, whhT, b_ih, b_hh):
        gi = jnp.dot(xt, wihT, preferred_element_type=jnp.float32) + b_ih
        gh = jnp.dot(h, whhT, preferred_element_type=jnp.float32) + b_hh
        i_r, i_z, i_n = gi[:, 0:H], gi[:, H:2 * H], gi[:, 2 * H:3 * H]
        h_r, h_z, h_n = gh[:, 0:H], gh[:, H:2 * H], gh[:, 2 * H:3 * H]
        r = jax.nn.sigmoid(i_r + h_r)
        z = jax.nn.sigmoid(i_z + h_z)
        n = jnp.tanh(i_n + r * h_n)
        return (1.0 - z) * n + z * h

    def body(i, carry):
        h_f, h_b = carry
        x_f = x_ref[pl.ds(i * B, B), :]
        x_b = x_ref[pl.ds((L - 1 - i) * B, B), :]
        h_f = cell(x_f, h_f, wihT_f, whhT_f, b_f, bh_f)
        h_b = cell(x_b, h_b, wihT_b, whhT_b, b_b, bh_b)
        out_ref[pl.ds(i * B, B), 0:H] = h_f
        out_ref[pl.ds((L - 3 - i) * B, B), H:2 * H] = h_b
        return h_f, h_b

    h0 = jnp.zeros((B, H), dtype=jnp.float32)
    jax.lax.fori_loop(0, L - 2, body, (h0, h0))


def _lse_kernel(x_ref, w_ref, b_ref, lse_ref, m_ref, s_ref):
    j = pl.program_id(0)

    @pl.when(j == 0)
    def _():
        m_ref[...] = jnp.full((NROWS, 1), NEG_INF, dtype=jnp.float32)
        s_ref[...] = jnp.zeros((NROWS, 1), dtype=jnp.float32)

    logits = jax.lax.dot_general(
        x_ref[...], w_ref[...], (((1,), (1,)), ((), ())),
        preferred_element_type=jnp.float32) + b_ref[...]
    col = j * VB + jax.lax.broadcasted_iota(jnp.int32, (NROWS, VB), 1)
    logits = jnp.where(col < V, logits, NEG_INF)

    m_old = m_ref[...]
    bmax = jnp.max(logits, axis=1, keepdims=True)
    m_new = jnp.maximum(m_old, bmax)
    bsum = jnp.sum(jnp.exp(logits - m_new), axis=1, keepdims=True)
    s_ref[...] = s_ref[...] * jnp.exp(m_old - m_new) + bsum
    m_ref[...] = m_new

    @pl.when(j == NVB - 1)
    def _():
        lse_ref[...] = m_ref[...] + jnp.log(s_ref[...])


def _logp_kernel(x_ref, w_ref, b_ref, lse_ref, out_ref):
    logits = jax.lax.dot_general(
        x_ref[...], w_ref[...], (((1,), (1,)), ((), ())),
        preferred_element_type=jnp.float32) + b_ref[...]
    out_ref[...] = logits - lse_ref[...]


def kernel(seq, seq_length, emb, w_ih_f, w_hh_f, b_ih_f, b_hh_f,
           w_ih_b, w_hh_b, b_ih_b, b_hh_b, dec_W, dec_b):
    x = jnp.take(emb, seq.reshape(-1), axis=0)

    out = pl.pallas_call(
        _gru_kernel,
        out_shape=jax.ShapeDtypeStruct((NROWS, 2 * H), jnp.float32),
    )(x, w_ih_f.T, w_hh_f.T, b_ih_f.reshape(1, -1), b_hh_f.reshape(1, -1),
      w_ih_b.T, w_hh_b.T, b_ih_b.reshape(1, -1), b_hh_b.reshape(1, -1))

    dec_b2 = dec_b.reshape(1, V)

    lse = pl.pallas_call(
        _lse_kernel,
        grid=(NVB,),
        in_specs=[
            pl.BlockSpec((NROWS, 2 * H), lambda j: (0, 0)),
            pl.BlockSpec((VB, 2 * H), lambda j: (j, 0)),
            pl.BlockSpec((1, VB), lambda j: (0, j)),
        ],
        out_specs=pl.BlockSpec((NROWS, 1), lambda j: (0, 0)),
        out_shape=jax.ShapeDtypeStruct((NROWS, 1), jnp.float32),
        scratch_shapes=[
            pltpu.VMEM((NROWS, 1), jnp.float32),
            pltpu.VMEM((NROWS, 1), jnp.float32),
        ],
    )(out, dec_W, dec_b2)

    logp = pl.pallas_call(
        _logp_kernel,
        grid=(NVB,),
        in_specs=[
            pl.BlockSpec((NROWS, 2 * H), lambda j: (0, 0)),
            pl.BlockSpec((VB, 2 * H), lambda j: (j, 0)),
            pl.BlockSpec((1, VB), lambda j: (0, j)),
            pl.BlockSpec((NROWS, 1), lambda j: (0, 0)),
        ],
        out_specs=pl.BlockSpec((NROWS, VB), lambda j: (0, j)),
        out_shape=jax.ShapeDtypeStruct((NROWS, V), jnp.float32),
    )(out, dec_W, dec_b2, lse)

    return logp.reshape(L - 2, B, V)
